# speedup vs baseline: 19.1722x; 19.1722x over previous
import functools

import jax
import jax.numpy as jnp
from jax import lax
from jax.experimental import pallas as pl
from jax.experimental.pallas import tpu as pltpu
from jax.experimental.pallas import tpu_sc as plsc

N = 10000
NPAD = 10240
D = 128
E = 320000
G = 64
NCORE = 2
NSUB = 16
NW = NCORE * NSUB
EW = (E + NW * 128 - 1) // (NW * 128) * 128
CH = 128
NCHUNK = EW // CH
RPT = NPAD // NSUB
BN = 512
GRID = NPAD // BN

_sc_mesh = plsc.VectorSubcoreMesh(core_axis_name="c", subcore_axis_name="s")


@functools.partial(
    pl.kernel,
    out_type=jax.ShapeDtypeStruct((NCORE, NPAD), jnp.float32),
    mesh=_sc_mesh,
    scratch_types=[
        pltpu.VMEM((NCHUNK, CH), jnp.int32),
        pltpu.VMEM((CH,), jnp.float32),
        pltpu.VMEM_SHARED((NPAD,), jnp.float32),
    ],
)
def _deg_sc(dst_hbm, zeros_hbm, out_hbm, dst_v, ones_v, acc_sh):
    c = lax.axis_index("c")
    s = lax.axis_index("s")
    wid = c * NSUB + s
    pltpu.sync_copy(dst_hbm.at[wid], dst_v)
    for j in range(CH // 16):
        ones_v[pl.ds(j * 16, 16)] = jnp.ones((16,), jnp.float32)
    pltpu.sync_copy(zeros_hbm.at[pl.ds(s * RPT, RPT)], acc_sh.at[pl.ds(s * RPT, RPT)])
    plsc.subcore_barrier()

    def body(j, carry):
        pltpu.sync_copy(ones_v, acc_sh.at[dst_v.at[j]], add=True)
        return carry

    lax.fori_loop(0, NCHUNK, body, 0)
    plsc.subcore_barrier()
    pltpu.sync_copy(acc_sh.at[pl.ds(s * RPT, RPT)], out_hbm.at[c, pl.ds(s * RPT, RPT)])


@functools.partial(
    pl.kernel,
    out_type=jax.ShapeDtypeStruct((NCORE, NPAD, D), jnp.float32),
    mesh=_sc_mesh,
    scratch_types=[
        pltpu.VMEM((NCHUNK, CH), jnp.int32),
        pltpu.VMEM((NCHUNK, CH), jnp.int32),
        pltpu.VMEM((CH, D), jnp.float32),
        pltpu.VMEM_SHARED((NPAD, D), jnp.float32),
    ],
)
def _prop_sc(mt_hbm, src_hbm, dst_hbm, zeros_hbm, out_hbm, src_v, dst_v, buf, acc_sh):
    c = lax.axis_index("c")
    s = lax.axis_index("s")
    wid = c * NSUB + s
    pltpu.sync_copy(src_hbm.at[wid], src_v)
    pltpu.sync_copy(dst_hbm.at[wid], dst_v)
    pltpu.sync_copy(zeros_hbm.at[pl.ds(s * RPT, RPT)], acc_sh.at[pl.ds(s * RPT, RPT)])
    plsc.subcore_barrier()

    def body(j, carry):
        pltpu.sync_copy(mt_hbm.at[src_v.at[j]], buf)
        pltpu.sync_copy(buf, acc_sh.at[dst_v.at[j]], add=True)
        return carry

    lax.fori_loop(0, NCHUNK, body, 0)
    plsc.subcore_barrier()
    pltpu.sync_copy(acc_sh.at[pl.ds(s * RPT, RPT)], out_hbm.at[c, pl.ds(s * RPT, RPT)])


def _gates_body(x_ref, p0_ref, p1_ref, win_ref, bin_ref, iww0_ref, ww0_ref,
                fww1_ref, iww1_ref, ww1_ref, tb0_ref, fwu1_ref, iwu1_ref,
                mt0_ref, theta_ref, if1_ref, dinv_ref):
    i = pl.program_id(0)
    f32 = jnp.float32
    x = x_ref[...]
    h = jnp.dot(x, win_ref[...], preferred_element_type=f32) + bin_ref[...]
    indeg = p0_ref[...] + p1_ref[...]
    dinv = lax.rsqrt(indeg + 1.0)
    rows = i * BN + lax.broadcasted_iota(jnp.int32, (BN, 1), 0)
    mask = (rows < N).astype(f32)
    dinv = mask * dinv
    vf = jnp.dot(tb0_ref[...], fwu1_ref[...], preferred_element_type=f32)
    vi = jnp.dot(tb0_ref[...], iwu1_ref[...], preferred_element_type=f32)
    m0 = jax.nn.sigmoid(
        jax.nn.sigmoid(jnp.dot(h, iww0_ref[...], preferred_element_type=f32))
        * jnp.tanh(jnp.dot(h, ww0_ref[...], preferred_element_type=f32)))
    mt0_ref[...] = dinv * m0
    theta_ref[...] = jax.nn.sigmoid(
        indeg * vf + jnp.dot(h, fww1_ref[...], preferred_element_type=f32))
    if1_ref[...] = jax.nn.sigmoid(
        indeg * vi + jnp.dot(h, iww1_ref[...], preferred_element_type=f32)
    ) * jnp.tanh(jnp.dot(h, ww1_ref[...], preferred_element_type=f32))
    dinv_ref[...] = dinv


def _mid_body(a0_ref, a1_ref, mt0_ref, dinv_ref, theta_ref, if1_ref, mt2_ref):
    dinv = dinv_ref[...]
    m1 = dinv * (a0_ref[...] + a1_ref[...] + mt0_ref[...])
    m2 = jax.nn.sigmoid(m1 * theta_ref[...] + if1_ref[...])
    mt2_ref[...] = dinv * m2


def _epi_body(b0_ref, b1_ref, mt2_ref, dinv_ref, bid_ref, wcls_ref, bcls_ref,
              out_ref, s_ref, t_ref, c_ref):
    i = pl.program_id(0)
    f32 = jnp.float32

    @pl.when(i == 0)
    def _():
        s_ref[...] = jnp.zeros_like(s_ref)
        t_ref[...] = jnp.zeros_like(t_ref)
        c_ref[...] = jnp.zeros_like(c_ref)

    m3 = dinv_ref[...] * (b0_ref[...] + b1_ref[...] + mt2_ref[...])
    mx = jnp.max(m3, axis=1, keepdims=True)
    ex = jnp.exp(m3 - mx)
    lse = mx + jnp.log(jnp.sum(ex, axis=1, keepdims=True))
    onehot = (bid_ref[...] == lax.broadcasted_iota(jnp.int32, (BN, G), 1)).astype(f32)
    dn = (((0,), (0,)), ((), ()))
    s_ref[...] += lax.dot_general(onehot, m3, dn, preferred_element_type=f32)
    t_ref[...] += lax.dot_general(onehot, lse, dn, preferred_element_type=f32)
    c_ref[...] += lax.dot_general(onehot, jnp.ones((BN, 1), f32), dn,
                                  preferred_element_type=f32)

    @pl.when(i == GRID - 1)
    def _():
        wcls = wcls_ref[...]
        colsum = jnp.sum(wcls, axis=0, keepdims=True)
        out = jnp.dot(s_ref[...], wcls, preferred_element_type=f32) - t_ref[...] * colsum
        out_ref[...] = out / jnp.maximum(c_ref[...], 1.0) + bcls_ref[...]


def _row_spec(w):
    return pl.BlockSpec((BN, w), lambda i: (i, 0))


def _full_spec(shape):
    return pl.BlockSpec(shape, lambda i: tuple(0 for _ in shape))


_gates_call = pl.pallas_call(
    _gates_body,
    grid=(GRID,),
    in_specs=[
        _row_spec(D), _row_spec(1), _row_spec(1),
        _full_spec((D, D)), _full_spec((1, D)),
        _full_spec((D, D)), _full_spec((D, D)), _full_spec((D, D)),
        _full_spec((D, D)), _full_spec((D, D)),
        _full_spec((1, D)), _full_spec((D, D)), _full_spec((D, D)),
    ],
    out_specs=[_row_spec(D), _row_spec(D), _row_spec(D), _row_spec(1)],
    out_shape=[
        jax.ShapeDtypeStruct((NPAD, D), jnp.float32),
        jax.ShapeDtypeStruct((NPAD, D), jnp.float32),
        jax.ShapeDtypeStruct((NPAD, D), jnp.float32),
        jax.ShapeDtypeStruct((NPAD, 1), jnp.float32),
    ],
)

_mid_call = pl.pallas_call(
    _mid_body,
    grid=(GRID,),
    in_specs=[_row_spec(D), _row_spec(D), _row_spec(D), _row_spec(1),
              _row_spec(D), _row_spec(D)],
    out_specs=_row_spec(D),
    out_shape=jax.ShapeDtypeStruct((NPAD, D), jnp.float32),
)

_epi_call = pl.pallas_call(
    _epi_body,
    grid=(GRID,),
    in_specs=[_row_spec(D), _row_spec(D), _row_spec(D), _row_spec(1),
              _row_spec(1), _full_spec((D, D)), _full_spec((1, D))],
    out_specs=_full_spec((G, D)),
    out_shape=jax.ShapeDtypeStruct((G, D), jnp.float32),
    scratch_shapes=[
        pltpu.VMEM((G, D), jnp.float32),
        pltpu.VMEM((G, 1), jnp.float32),
        pltpu.VMEM((G, 1), jnp.float32),
    ],
)


def kernel(x, edge_index, batch_ids, W_in, b_in, f_Ww, f_Wu, i_Ww, i_Wu, Ww,
           t_W, t_b, W_cls, b_cls):
    src = edge_index[0].astype(jnp.int32)
    dst = edge_index[1].astype(jnp.int32)
    pad_e = NW * EW - E
    src_p = jnp.concatenate([src, jnp.full((pad_e,), N, jnp.int32)]).reshape(NW, NCHUNK, CH)
    dst_p = jnp.concatenate([dst, jnp.full((pad_e,), NPAD - 1, jnp.int32)]).reshape(NW, NCHUNK, CH)
    x_p = jnp.pad(x, ((0, NPAD - N), (0, 0)))
    bid_p = jnp.pad(batch_ids.astype(jnp.int32), (0, NPAD - N),
                    constant_values=G + 7).reshape(NPAD, 1)
    zeros1 = jnp.zeros((NPAD,), jnp.float32)
    zerosn = jnp.zeros((NPAD, D), jnp.float32)

    degp = _deg_sc(dst_p, zeros1)
    p0 = degp[0].reshape(NPAD, 1)
    p1 = degp[1].reshape(NPAD, 1)
    mt0, theta, if1, dinv = _gates_call(
        x_p, p0, p1, W_in, b_in.reshape(1, D), i_Ww[0], Ww[0], f_Ww[1],
        i_Ww[1], Ww[1], t_b[0].reshape(1, D), f_Wu[1], i_Wu[1])
    acc1 = _prop_sc(mt0, src_p, dst_p, zerosn)
    mt2 = _mid_call(acc1[0], acc1[1], mt0, dinv, theta, if1)
    acc2 = _prop_sc(mt2, src_p, dst_p, zerosn)
    out = _epi_call(acc2[0], acc2[1], mt2, dinv, bid_p, W_cls, b_cls.reshape(1, D))
    return out


# speedup vs baseline: 26.1420x; 1.3635x over previous
import functools

import jax
import jax.numpy as jnp
from jax import lax
from jax.experimental import pallas as pl
from jax.experimental.pallas import tpu as pltpu
from jax.experimental.pallas import tpu_sc as plsc

N = 10000
D = 128
E = 320000
G = 64
NCORE = 2
NSUB = 16
NW = NCORE * NSUB
ACCROWS = 10240
RPT = ACCROWS // NSUB

CH = 64
DEPTH = 4
EW = (E + NW * CH - 1) // (NW * CH) * CH
NCHUNK = EW // CH
MTROWS = 10240

CHD = 100
NCHD = E // (NW * CHD)
NDEG = 10240
RPTD = NDEG // NSUB

BN = 400
GRID = N // BN

_sc_mesh = plsc.VectorSubcoreMesh(core_axis_name="c", subcore_axis_name="s")


@functools.partial(
    pl.kernel,
    out_type=jax.ShapeDtypeStruct((NCORE, NDEG), jnp.float32),
    mesh=_sc_mesh,
    scratch_types=[
        pltpu.VMEM((NCHD, CHD), jnp.int32),
        pltpu.VMEM((CHD,), jnp.float32),
        pltpu.VMEM_SHARED((NDEG,), jnp.float32),
    ],
)
def _deg_sc(dst_hbm, zeros_hbm, ones_hbm, out_hbm, dst_v, ones_v, acc_sh):
    c = lax.axis_index("c")
    s = lax.axis_index("s")
    wid = c * NSUB + s
    pltpu.sync_copy(dst_hbm.at[wid], dst_v)
    pltpu.sync_copy(ones_hbm, ones_v)
    pltpu.sync_copy(zeros_hbm.at[pl.ds(s * RPTD, RPTD)], acc_sh.at[pl.ds(s * RPTD, RPTD)])
    plsc.subcore_barrier()

    def body(j, carry):
        pltpu.sync_copy(ones_v, acc_sh.at[dst_v.at[j]], add=True)
        return carry

    lax.fori_loop(0, NCHD, body, 0)
    plsc.subcore_barrier()
    pltpu.sync_copy(acc_sh.at[pl.ds(s * RPTD, RPTD)], out_hbm.at[c, pl.ds(s * RPTD, RPTD)])


@functools.partial(
    pl.kernel,
    out_type=jax.ShapeDtypeStruct((NCORE, ACCROWS, D), jnp.float32),
    mesh=_sc_mesh,
    scratch_types=[
        pltpu.VMEM((DEPTH, 2, CH), jnp.int32),
        pltpu.VMEM((DEPTH, CH, D), jnp.float32),
        pltpu.VMEM_SHARED((ACCROWS, D), jnp.float32),
        [pltpu.SemaphoreType.DMA] * DEPTH,
        [pltpu.SemaphoreType.DMA] * DEPTH,
        [pltpu.SemaphoreType.DMA] * DEPTH,
    ],
)
def _prop_sc(mt_hbm, eg_hbm, zeros_hbm, out_hbm, idx_v, bufs, acc_sh,
             sem_i, sem_g, sem_s):
    c = lax.axis_index("c")
    s = lax.axis_index("s")
    wid = c * NSUB + s
    pltpu.sync_copy(zeros_hbm.at[pl.ds(s * RPT, RPT)], acc_sh.at[pl.ds(s * RPT, RPT)])
    plsc.subcore_barrier()

    def idx_cp(j, t):
        pltpu.async_copy(eg_hbm.at[wid, j], idx_v.at[t], sem_i[t])

    def idx_wait(j, t):
        pltpu.make_async_copy(eg_hbm.at[wid, j], idx_v.at[t], sem_i[t]).wait()

    def gather(t):
        pltpu.async_copy(mt_hbm.at[idx_v.at[t, 0]], bufs.at[t], sem_g[t])

    def gather_wait(t):
        pltpu.make_async_copy(mt_hbm.at[idx_v.at[t, 0]], bufs.at[t], sem_g[t]).wait()

    def scat(t):
        pltpu.async_copy(bufs.at[t], acc_sh.at[idx_v.at[t, 1]], sem_s[t], add=True)

    def scat_wait(t):
        pltpu.make_async_copy(bufs.at[t], acc_sh.at[idx_v.at[t, 1]], sem_s[t]).wait()

    for t in range(DEPTH):
        idx_cp(t, t)
    for t in range(DEPTH):
        idx_wait(t, t)
        gather(t)

    NITER = NCHUNK // DEPTH

    def body(k, carry):
        j = DEPTH * k
        for t in range(DEPTH):
            gather_wait(t)
            scat(t)

        @pl.when(k + 1 < NITER)
        def _():
            for t in range(DEPTH):
                scat_wait(t)
                idx_cp(j + DEPTH + t, t)
            for t in range(DEPTH):
                idx_wait(j + DEPTH + t, t)
                gather(t)

        return carry

    lax.fori_loop(0, NITER, body, 0)
    for t in range(DEPTH):
        scat_wait(t)
    plsc.subcore_barrier()
    pltpu.sync_copy(acc_sh.at[pl.ds(s * RPT, RPT)], out_hbm.at[c, pl.ds(s * RPT, RPT)])


def _gates_body(x_ref, p0_ref, p1_ref, win_ref, bin_ref, iww0_ref, ww0_ref,
                fww1_ref, iww1_ref, ww1_ref, tb0_ref, fwu1_ref, iwu1_ref,
                mt0_ref, theta_ref, if1_ref, dinv_ref):
    f32 = jnp.float32
    x = x_ref[...]
    h = jnp.dot(x, win_ref[...], preferred_element_type=f32) + bin_ref[...]
    indeg = p0_ref[...] + p1_ref[...]
    dinv = lax.rsqrt(indeg + 1.0)
    vf = jnp.dot(tb0_ref[...], fwu1_ref[...], preferred_element_type=f32)
    vi = jnp.dot(tb0_ref[...], iwu1_ref[...], preferred_element_type=f32)
    m0 = jax.nn.sigmoid(
        jax.nn.sigmoid(jnp.dot(h, iww0_ref[...], preferred_element_type=f32))
        * jnp.tanh(jnp.dot(h, ww0_ref[...], preferred_element_type=f32)))
    mt0_ref[...] = dinv * m0
    theta_ref[...] = jax.nn.sigmoid(
        indeg * vf + jnp.dot(h, fww1_ref[...], preferred_element_type=f32))
    if1_ref[...] = jax.nn.sigmoid(
        indeg * vi + jnp.dot(h, iww1_ref[...], preferred_element_type=f32)
    ) * jnp.tanh(jnp.dot(h, ww1_ref[...], preferred_element_type=f32))
    dinv_ref[...] = dinv


def _mid_body(a0_ref, a1_ref, mt0_ref, dinv_ref, theta_ref, if1_ref, mt2_ref):
    dinv = dinv_ref[...]
    m1 = dinv * (a0_ref[...] + a1_ref[...] + mt0_ref[...])
    m2 = jax.nn.sigmoid(m1 * theta_ref[...] + if1_ref[...])
    mt2_ref[...] = dinv * m2


def _epi_body(b0_ref, b1_ref, mt2_ref, dinv_ref, bid_ref, wcls_ref, bcls_ref,
              out_ref, s_ref, t_ref, c_ref):
    i = pl.program_id(0)
    f32 = jnp.float32

    @pl.when(i == 0)
    def _():
        s_ref[...] = jnp.zeros_like(s_ref)
        t_ref[...] = jnp.zeros_like(t_ref)
        c_ref[...] = jnp.zeros_like(c_ref)

    m3 = dinv_ref[...] * (b0_ref[...] + b1_ref[...] + mt2_ref[...])
    mx = jnp.max(m3, axis=1, keepdims=True)
    ex = jnp.exp(m3 - mx)
    lse = mx + jnp.log(jnp.sum(ex, axis=1, keepdims=True))
    onehot = (bid_ref[...] == lax.broadcasted_iota(jnp.int32, (BN, G), 1)).astype(f32)
    dn = (((0,), (0,)), ((), ()))
    s_ref[...] += lax.dot_general(onehot, m3, dn, preferred_element_type=f32)
    t_ref[...] += lax.dot_general(onehot, lse, dn, preferred_element_type=f32)
    c_ref[...] += lax.dot_general(onehot, jnp.ones((BN, 1), f32), dn,
                                  preferred_element_type=f32)

    @pl.when(i == GRID - 1)
    def _():
        wcls = wcls_ref[...]
        colsum = jnp.sum(wcls, axis=0, keepdims=True)
        out = jnp.dot(s_ref[...], wcls, preferred_element_type=f32) - t_ref[...] * colsum
        out_ref[...] = out / jnp.maximum(c_ref[...], 1.0) + bcls_ref[...]


def _row_spec(w):
    return pl.BlockSpec((BN, w), lambda i: (i, 0))


def _full_spec(shape):
    return pl.BlockSpec(shape, lambda i: tuple(0 for _ in shape))


_gates_call = pl.pallas_call(
    _gates_body,
    grid=(GRID,),
    in_specs=[
        _row_spec(D), _row_spec(1), _row_spec(1),
        _full_spec((D, D)), _full_spec((1, D)),
        _full_spec((D, D)), _full_spec((D, D)), _full_spec((D, D)),
        _full_spec((D, D)), _full_spec((D, D)),
        _full_spec((1, D)), _full_spec((D, D)), _full_spec((D, D)),
    ],
    out_specs=[_row_spec(D), _row_spec(D), _row_spec(D), _row_spec(1)],
    out_shape=[
        jax.ShapeDtypeStruct((N, D), jnp.float32),
        jax.ShapeDtypeStruct((N, D), jnp.float32),
        jax.ShapeDtypeStruct((N, D), jnp.float32),
        jax.ShapeDtypeStruct((N, 1), jnp.float32),
    ],
)

_mid_call = pl.pallas_call(
    _mid_body,
    grid=(GRID,),
    in_specs=[_row_spec(D), _row_spec(D), _row_spec(D), _row_spec(1),
              _row_spec(D), _row_spec(D)],
    out_specs=_row_spec(D),
    out_shape=jax.ShapeDtypeStruct((N, D), jnp.float32),
)

_epi_call = pl.pallas_call(
    _epi_body,
    grid=(GRID,),
    in_specs=[_row_spec(D), _row_spec(D), _row_spec(D), _row_spec(1),
              _row_spec(1), _full_spec((D, D)), _full_spec((1, D))],
    out_specs=_full_spec((G, D)),
    out_shape=jax.ShapeDtypeStruct((G, D), jnp.float32),
    scratch_shapes=[
        pltpu.VMEM((G, D), jnp.float32),
        pltpu.VMEM((G, 1), jnp.float32),
        pltpu.VMEM((G, 1), jnp.float32),
    ],
)


def kernel(x, edge_index, batch_ids, W_in, b_in, f_Ww, f_Wu, i_Ww, i_Wu, Ww,
           t_W, t_b, W_cls, b_cls):
    src = edge_index[0].astype(jnp.int32)
    dst = edge_index[1].astype(jnp.int32)
    pad_e = NW * EW - E
    src_p = jnp.concatenate([src, jnp.full((pad_e,), N, jnp.int32)]).reshape(NW, NCHUNK, CH)
    dst_p = jnp.concatenate([dst, jnp.full((pad_e,), ACCROWS - 1, jnp.int32)]).reshape(NW, NCHUNK, CH)
    eg = jnp.stack([src_p, dst_p], axis=2)
    dst_d = dst.reshape(NW, NCHD, CHD)
    bid = batch_ids.astype(jnp.int32).reshape(N, 1)
    zeros1 = jnp.zeros((NDEG,), jnp.float32)
    zerosn = jnp.zeros((ACCROWS, D), jnp.float32)

    degp = _deg_sc(dst_d, zeros1, jnp.ones((CHD,), jnp.float32))
    p0 = degp[0, :N].reshape(N, 1)
    p1 = degp[1, :N].reshape(N, 1)
    mt0, theta, if1, dinv = _gates_call(
        x, p0, p1, W_in, b_in.reshape(1, D), i_Ww[0], Ww[0], f_Ww[1],
        i_Ww[1], Ww[1], t_b[0].reshape(1, D), f_Wu[1], i_Wu[1])
    mt0_p = jnp.pad(mt0, ((0, MTROWS - N), (0, 0)))
    acc1 = _prop_sc(mt0_p, eg, zerosn)
    mt2 = _mid_call(acc1[0, :N], acc1[1, :N], mt0, dinv, theta, if1)
    mt2_p = jnp.pad(mt2, ((0, MTROWS - N), (0, 0)))
    acc2 = _prop_sc(mt2_p, eg, zerosn)
    out = _epi_call(acc2[0, :N], acc2[1, :N], mt2, dinv, bid, W_cls, b_cls.reshape(1, D))
    return out


# speedup vs baseline: 26.8639x; 1.0276x over previous
import functools

import jax
import jax.numpy as jnp
from jax import lax
from jax.experimental import pallas as pl
from jax.experimental.pallas import tpu as pltpu
from jax.experimental.pallas import tpu_sc as plsc

N = 10000
D = 128
E = 320000
G = 64
NCORE = 2
NSUB = 16
NW = NCORE * NSUB
ACCROWS = 10240
RPT = ACCROWS // NSUB

CH = 64
DEPTH = 5
EW = (E + NW * CH - 1) // (NW * CH) * CH
NCHUNK = EW // CH
MTROWS = 10240

CHD = 100
NCHD = E // (NW * CHD)
NDEG = 10240
RPTD = NDEG // NSUB

BN = 400
GRID = N // BN

_sc_mesh = plsc.VectorSubcoreMesh(core_axis_name="c", subcore_axis_name="s")


@functools.partial(
    pl.kernel,
    out_type=jax.ShapeDtypeStruct((NCORE, NDEG), jnp.float32),
    mesh=_sc_mesh,
    scratch_types=[
        pltpu.VMEM((NCHD, CHD), jnp.int32),
        pltpu.VMEM((CHD,), jnp.float32),
        pltpu.VMEM_SHARED((NDEG,), jnp.float32),
    ],
)
def _deg_sc(dst_hbm, zeros_hbm, ones_hbm, out_hbm, dst_v, ones_v, acc_sh):
    c = lax.axis_index("c")
    s = lax.axis_index("s")
    wid = c * NSUB + s
    pltpu.sync_copy(dst_hbm.at[wid], dst_v)
    pltpu.sync_copy(ones_hbm, ones_v)
    pltpu.sync_copy(zeros_hbm.at[pl.ds(s * RPTD, RPTD)], acc_sh.at[pl.ds(s * RPTD, RPTD)])
    plsc.subcore_barrier()

    def body(j, carry):
        pltpu.sync_copy(ones_v, acc_sh.at[dst_v.at[j]], add=True)
        return carry

    lax.fori_loop(0, NCHD, body, 0)
    plsc.subcore_barrier()
    pltpu.sync_copy(acc_sh.at[pl.ds(s * RPTD, RPTD)], out_hbm.at[c, pl.ds(s * RPTD, RPTD)])


@functools.partial(
    pl.kernel,
    out_type=jax.ShapeDtypeStruct((NCORE, ACCROWS, D), jnp.float32),
    mesh=_sc_mesh,
    scratch_types=[
        pltpu.VMEM((DEPTH, 2, CH), jnp.int32),
        pltpu.VMEM((DEPTH, CH, D), jnp.float32),
        pltpu.VMEM_SHARED((ACCROWS, D), jnp.float32),
        [pltpu.SemaphoreType.DMA] * DEPTH,
        [pltpu.SemaphoreType.DMA] * DEPTH,
        [pltpu.SemaphoreType.DMA] * DEPTH,
    ],
)
def _prop_sc(mt_hbm, eg_hbm, zeros_hbm, out_hbm, idx_v, bufs, acc_sh,
             sem_i, sem_g, sem_s):
    c = lax.axis_index("c")
    s = lax.axis_index("s")
    wid = c * NSUB + s
    pltpu.sync_copy(zeros_hbm.at[pl.ds(s * RPT, RPT)], acc_sh.at[pl.ds(s * RPT, RPT)])
    plsc.subcore_barrier()

    def idx_cp(j, t):
        pltpu.async_copy(eg_hbm.at[wid, j], idx_v.at[t], sem_i[t])

    def idx_wait(j, t):
        pltpu.make_async_copy(eg_hbm.at[wid, j], idx_v.at[t], sem_i[t]).wait()

    def gather(t):
        pltpu.async_copy(mt_hbm.at[idx_v.at[t, 0]], bufs.at[t], sem_g[t])

    def gather_wait(t):
        pltpu.make_async_copy(mt_hbm.at[idx_v.at[t, 0]], bufs.at[t], sem_g[t]).wait()

    def scat(t):
        pltpu.async_copy(bufs.at[t], acc_sh.at[idx_v.at[t, 1]], sem_s[t], add=True)

    def scat_wait(t):
        pltpu.make_async_copy(bufs.at[t], acc_sh.at[idx_v.at[t, 1]], sem_s[t]).wait()

    for t in range(DEPTH):
        idx_cp(t, t)
    for t in range(DEPTH):
        idx_wait(t, t)
        gather(t)

    NITER = NCHUNK // DEPTH

    def body(k, carry):
        j = DEPTH * k
        for t in range(DEPTH):
            gather_wait(t)
            scat(t)

        @pl.when(k + 1 < NITER)
        def _():
            for t in range(DEPTH):
                scat_wait(t)
                idx_cp(j + DEPTH + t, t)
            for t in range(DEPTH):
                idx_wait(j + DEPTH + t, t)
                gather(t)

        return carry

    lax.fori_loop(0, NITER, body, 0)
    for t in range(DEPTH):
        scat_wait(t)
    plsc.subcore_barrier()
    pltpu.sync_copy(acc_sh.at[pl.ds(s * RPT, RPT)], out_hbm.at[c, pl.ds(s * RPT, RPT)])


def _gates_body(x_ref, p0_ref, p1_ref, win_ref, bin_ref, iww0_ref, ww0_ref,
                fww1_ref, iww1_ref, ww1_ref, tb0_ref, fwu1_ref, iwu1_ref,
                mt0_ref, theta_ref, if1_ref, dinv_ref):
    f32 = jnp.float32
    x = x_ref[...]
    h = jnp.dot(x, win_ref[...], preferred_element_type=f32) + bin_ref[...]
    indeg = p0_ref[...] + p1_ref[...]
    dinv = lax.rsqrt(indeg + 1.0)
    vf = jnp.dot(tb0_ref[...], fwu1_ref[...], preferred_element_type=f32)
    vi = jnp.dot(tb0_ref[...], iwu1_ref[...], preferred_element_type=f32)
    m0 = jax.nn.sigmoid(
        jax.nn.sigmoid(jnp.dot(h, iww0_ref[...], preferred_element_type=f32))
        * jnp.tanh(jnp.dot(h, ww0_ref[...], preferred_element_type=f32)))
    mt0_ref[...] = dinv * m0
    theta_ref[...] = jax.nn.sigmoid(
        indeg * vf + jnp.dot(h, fww1_ref[...], preferred_element_type=f32))
    if1_ref[...] = jax.nn.sigmoid(
        indeg * vi + jnp.dot(h, iww1_ref[...], preferred_element_type=f32)
    ) * jnp.tanh(jnp.dot(h, ww1_ref[...], preferred_element_type=f32))
    dinv_ref[...] = dinv


def _mid_body(a0_ref, a1_ref, mt0_ref, dinv_ref, theta_ref, if1_ref, mt2_ref):
    dinv = dinv_ref[...]
    m1 = dinv * (a0_ref[...] + a1_ref[...] + mt0_ref[...])
    m2 = jax.nn.sigmoid(m1 * theta_ref[...] + if1_ref[...])
    mt2_ref[...] = dinv * m2


def _epi_body(b0_ref, b1_ref, mt2_ref, dinv_ref, bid_ref, wcls_ref, bcls_ref,
              out_ref, s_ref, t_ref, c_ref):
    i = pl.program_id(0)
    f32 = jnp.float32

    @pl.when(i == 0)
    def _():
        s_ref[...] = jnp.zeros_like(s_ref)
        t_ref[...] = jnp.zeros_like(t_ref)
        c_ref[...] = jnp.zeros_like(c_ref)

    m3 = dinv_ref[...] * (b0_ref[...] + b1_ref[...] + mt2_ref[...])
    mx = jnp.max(m3, axis=1, keepdims=True)
    ex = jnp.exp(m3 - mx)
    lse = mx + jnp.log(jnp.sum(ex, axis=1, keepdims=True))
    onehot = (bid_ref[...] == lax.broadcasted_iota(jnp.int32, (BN, G), 1)).astype(f32)
    dn = (((0,), (0,)), ((), ()))
    s_ref[...] += lax.dot_general(onehot, m3, dn, preferred_element_type=f32)
    t_ref[...] += lax.dot_general(onehot, lse, dn, preferred_element_type=f32)
    c_ref[...] += lax.dot_general(onehot, jnp.ones((BN, 1), f32), dn,
                                  preferred_element_type=f32)

    @pl.when(i == GRID - 1)
    def _():
        wcls = wcls_ref[...]
        colsum = jnp.sum(wcls, axis=0, keepdims=True)
        out = jnp.dot(s_ref[...], wcls, preferred_element_type=f32) - t_ref[...] * colsum
        out_ref[...] = out / jnp.maximum(c_ref[...], 1.0) + bcls_ref[...]


def _row_spec(w):
    return pl.BlockSpec((BN, w), lambda i: (i, 0))


def _full_spec(shape):
    return pl.BlockSpec(shape, lambda i: tuple(0 for _ in shape))


_gates_call = pl.pallas_call(
    _gates_body,
    grid=(GRID,),
    in_specs=[
        _row_spec(D), _row_spec(1), _row_spec(1),
        _full_spec((D, D)), _full_spec((1, D)),
        _full_spec((D, D)), _full_spec((D, D)), _full_spec((D, D)),
        _full_spec((D, D)), _full_spec((D, D)),
        _full_spec((1, D)), _full_spec((D, D)), _full_spec((D, D)),
    ],
    out_specs=[_row_spec(D), _row_spec(D), _row_spec(D), _row_spec(1)],
    out_shape=[
        jax.ShapeDtypeStruct((N, D), jnp.float32),
        jax.ShapeDtypeStruct((N, D), jnp.float32),
        jax.ShapeDtypeStruct((N, D), jnp.float32),
        jax.ShapeDtypeStruct((N, 1), jnp.float32),
    ],
)

_mid_call = pl.pallas_call(
    _mid_body,
    grid=(GRID,),
    in_specs=[_row_spec(D), _row_spec(D), _row_spec(D), _row_spec(1),
              _row_spec(D), _row_spec(D)],
    out_specs=_row_spec(D),
    out_shape=jax.ShapeDtypeStruct((N, D), jnp.float32),
)

_epi_call = pl.pallas_call(
    _epi_body,
    grid=(GRID,),
    in_specs=[_row_spec(D), _row_spec(D), _row_spec(D), _row_spec(1),
              _row_spec(1), _full_spec((D, D)), _full_spec((1, D))],
    out_specs=_full_spec((G, D)),
    out_shape=jax.ShapeDtypeStruct((G, D), jnp.float32),
    scratch_shapes=[
        pltpu.VMEM((G, D), jnp.float32),
        pltpu.VMEM((G, 1), jnp.float32),
        pltpu.VMEM((G, 1), jnp.float32),
    ],
)


def kernel(x, edge_index, batch_ids, W_in, b_in, f_Ww, f_Wu, i_Ww, i_Wu, Ww,
           t_W, t_b, W_cls, b_cls):
    src = edge_index[0].astype(jnp.int32)
    dst = edge_index[1].astype(jnp.int32)
    pad_e = NW * EW - E
    src_p = jnp.concatenate([src, jnp.full((pad_e,), N, jnp.int32)]).reshape(NW, NCHUNK, CH)
    dst_p = jnp.concatenate([dst, jnp.full((pad_e,), ACCROWS - 1, jnp.int32)]).reshape(NW, NCHUNK, CH)
    eg = jnp.stack([src_p, dst_p], axis=2)
    dst_d = dst.reshape(NW, NCHD, CHD)
    bid = batch_ids.astype(jnp.int32).reshape(N, 1)
    zeros1 = jnp.zeros((NDEG,), jnp.float32)
    zerosn = jnp.zeros((ACCROWS, D), jnp.float32)

    degp = _deg_sc(dst_d, zeros1, jnp.ones((CHD,), jnp.float32))
    p0 = degp[0, :N].reshape(N, 1)
    p1 = degp[1, :N].reshape(N, 1)
    mt0, theta, if1, dinv = _gates_call(
        x, p0, p1, W_in, b_in.reshape(1, D), i_Ww[0], Ww[0], f_Ww[1],
        i_Ww[1], Ww[1], t_b[0].reshape(1, D), f_Wu[1], i_Wu[1])
    mt0_p = jnp.pad(mt0, ((0, MTROWS - N), (0, 0)))
    acc1 = _prop_sc(mt0_p, eg, zerosn)
    mt2 = _mid_call(acc1[0, :N], acc1[1, :N], mt0, dinv, theta, if1)
    mt2_p = jnp.pad(mt2, ((0, MTROWS - N), (0, 0)))
    acc2 = _prop_sc(mt2_p, eg, zerosn)
    out = _epi_call(acc2[0, :N], acc2[1, :N], mt2, dinv, bid, W_cls, b_cls.reshape(1, D))
    return out
